# baseline (speedup 1.0000x reference)
import numpy as np
import jax
import jax.numpy as jnp
from jax.experimental import pallas as pl
from jax.experimental.pallas import tpu as pltpu

_NUM_HEADS = 12
_KEY_DIM = 384
_HEAD_DIM = _KEY_DIM // _NUM_HEADS
_BLOCK_SIZE = 32
_WINDOW_SIZE = 2
_NUM_RAND_BLOCKS = 2
_GLOBAL_TOKENS = 0
_D_MODEL = 768
_SEQ_LEN = 2048


def _bigbird_structure():
    seq_len, block_size = _SEQ_LEN, _BLOCK_SIZE
    num_blocks = (seq_len + block_size - 1) // block_size
    rows, cols = [], []
    for i in range(num_blocks):
        lo = max(0, i - _WINDOW_SIZE)
        hi = min(num_blocks, i + _WINDOW_SIZE + 1)
        for j in range(lo, hi):
            rows.append(i)
            cols.append(j)
    for i in range(num_blocks):
        for g in range(_GLOBAL_TOKENS):
            rows.append(i)
            cols.append(g)
    for g in range(_GLOBAL_TOKENS):
        for j in range(num_blocks):
            rows.append(g)
            cols.append(j)
    rng = np.random.default_rng(0)
    n_rand = _NUM_RAND_BLOCKS * num_blocks
    r1 = rng.integers(0, num_blocks, size=n_rand)
    r2 = rng.integers(0, num_blocks, size=n_rand)
    rows = np.concatenate([np.asarray(rows, dtype=np.int64), r1.astype(np.int64)])
    cols = np.concatenate([np.asarray(cols, dtype=np.int64), r2.astype(np.int64)])
    uniq = np.unique(rows * num_blocks + cols)
    ur = (uniq // num_blocks).astype(np.int64)
    uc = (uniq % num_blocks).astype(np.int64)
    nbr_lists = [uc[ur == b].astype(np.int32) for b in range(num_blocks)]
    max_nb = max(len(t) for t in nbr_lists)
    M = max_nb * block_size
    nbr_pad = np.zeros((num_blocks, max_nb), dtype=np.int32)
    bias = np.full((num_blocks, M), -1e9, dtype=np.float32)
    dense_idx = -np.ones((num_blocks, M), dtype=np.int32)
    for b, t in enumerate(nbr_lists):
        nbr_pad[b, : len(t)] = t
        bias[b, : len(t) * block_size] = 0.0
        if len(t):
            toks = np.concatenate(
                [np.arange(c * block_size, (c + 1) * block_size) for c in t])
            dense_idx[b, : len(toks)] = toks.astype(np.int32)
    token_block = (np.arange(seq_len) // block_size).astype(np.int64)
    attn_idx = dense_idx[token_block]
    return num_blocks, max_nb, M, nbr_pad, bias, attn_idx


_NUM_BLOCKS, _MAX_NB, _M, _NBR_PAD, _BIAS_NP, _ATTN_IDX_NP = _bigbird_structure()

_HEADS_PER_GROUP = 4
_NUM_GROUPS = _NUM_HEADS // _HEADS_PER_GROUP


_PROJ_STEPS = _SEQ_LEN // 256


def _mega_kernel(xq_ref, xk_ref, xv_ref, wq_ref, bq_ref, wk_ref, bk_ref,
                 wv_ref, bv_ref, bias_ref, wo_ref, bo_ref, scores_ref, out_ref,
                 q_sc, k_sc, v_sc):
    t = pl.program_id(0)

    @pl.when(t < _PROJ_STEPS)
    def _proj_phase():
        rows = pl.ds(t * 256, 256)
        qc = (jnp.dot(xq_ref[...], wq_ref[...], preferred_element_type=jnp.float32)
              + bq_ref[...]).astype(jnp.bfloat16)
        kc = (jnp.dot(xk_ref[...], wk_ref[...], preferred_element_type=jnp.float32)
              + bk_ref[...]).astype(jnp.bfloat16)
        vc = (jnp.dot(xv_ref[...], wv_ref[...], preferred_element_type=jnp.float32)
              + bv_ref[...]).astype(jnp.bfloat16)
        for hi in range(_NUM_HEADS):
            hl = slice(hi * _HEAD_DIM, (hi + 1) * _HEAD_DIM)
            q_sc[hi, rows, :] = qc[:, hl]
            k_sc[hi, rows, :] = kc[:, hl]
            v_sc[hi, rows, :] = vc[:, hl]

    @pl.when(t >= _PROJ_STEPS)
    def _attn_phase():
        h = t - _PROJ_STEPS
        q = q_sc[h]
        k = k_sc[h]
        v = v_sc[h]
        ps = []
        for i in range(_NUM_BLOCKS):
            nbrs = _NBR_PAD[i]
            rows = slice(i * _BLOCK_SIZE, (i + 1) * _BLOCK_SIZE)
            q_i = q[rows, :]
            kn = jnp.concatenate(
                [k[int(c) * _BLOCK_SIZE:(int(c) + 1) * _BLOCK_SIZE, :] for c in nbrs],
                axis=0)
            s = jax.lax.dot_general(q_i, kn, (((1,), (1,)), ((), ())),
                                    preferred_element_type=jnp.float32)
            s = s + bias_ref[i:i + 1, :]
            scores_ref[0, 0, rows, :] = s
            m = jnp.max(s, axis=-1, keepdims=True)
            e = jnp.exp(s - m)
            ps.append((e / jnp.sum(e, axis=-1, keepdims=True)).astype(jnp.bfloat16))
        ctx_parts = []
        for i in range(_NUM_BLOCKS):
            nbrs = _NBR_PAD[i]
            vn = jnp.concatenate(
                [v[int(c) * _BLOCK_SIZE:(int(c) + 1) * _BLOCK_SIZE, :] for c in nbrs],
                axis=0)
            ctx_parts.append(jax.lax.dot_general(ps[i], vn, (((1,), (0,)), ((), ())),
                                                 preferred_element_type=jnp.float32))
        q_sc[h] = jnp.concatenate(ctx_parts, axis=0).astype(jnp.bfloat16)

        @pl.when(h == _NUM_HEADS - 1)
        def _():
            acc = bo_ref[...].astype(jnp.float32)
            for hi in range(_NUM_HEADS):
                w_rows = slice(hi * _HEAD_DIM, (hi + 1) * _HEAD_DIM)
                acc = acc + jnp.dot(q_sc[hi], wo_ref[w_rows, :],
                                    preferred_element_type=jnp.float32)
            out_ref[0] = acc


def kernel(query, value, key_in, Wq, bq, Wk, bk, Wv, bv, Wo, bo):
    B, S, D = query.shape
    H, hd, M, NB = _NUM_HEADS, _HEAD_DIM, _M, _NUM_BLOCKS
    scale = float(hd) ** -0.5

    ROWS = 256
    GL = _HEADS_PER_GROUP * hd
    P = _PROJ_STEPS
    row_spec = pl.BlockSpec((ROWS, D), lambda t: (jnp.minimum(t, P - 1), 0))
    w_spec = pl.BlockSpec((D, _KEY_DIM), lambda t: (0, 0))
    b_spec = pl.BlockSpec((1, _KEY_DIM), lambda t: (0, 0))
    scores, out = pl.pallas_call(
        _mega_kernel,
        grid=(P + _NUM_HEADS,),
        in_specs=[row_spec, row_spec, row_spec,
                  w_spec, b_spec, w_spec, b_spec, w_spec, b_spec,
                  pl.BlockSpec((NB, M), lambda t: (0, 0)),
                  pl.BlockSpec((_KEY_DIM, D), lambda t: (0, 0)),
                  pl.BlockSpec((1, D), lambda t: (0, 0))],
        out_specs=[pl.BlockSpec(
                       (1, 1, S, M),
                       lambda t: (0, jnp.clip(t - P, 0, _NUM_HEADS - 1), 0, 0)),
                   pl.BlockSpec((1, S, D), lambda t: (0, 0, 0))],
        out_shape=[jax.ShapeDtypeStruct((1, H, S, M), jnp.float32),
                   jax.ShapeDtypeStruct((1, S, D), jnp.float32)],
        scratch_shapes=[pltpu.VMEM((_NUM_HEADS, S, hd), jnp.bfloat16),
                        pltpu.VMEM((_NUM_HEADS, S, hd), jnp.bfloat16),
                        pltpu.VMEM((_NUM_HEADS, S, hd), jnp.bfloat16)],
    )(query[0], key_in[0], value[0],
      Wq * scale, (bq * scale)[None, :], Wk, bk[None, :], Wv, bv[None, :],
      jnp.asarray(_BIAS_NP), Wo.astype(jnp.bfloat16), bo[None, :])

    return (out, scores, jnp.asarray(_ATTN_IDX_NP))


# speedup vs baseline: 1.0694x; 1.0694x over previous
import numpy as np
import jax
import jax.numpy as jnp
from jax.experimental import pallas as pl
from jax.experimental.pallas import tpu as pltpu

_NUM_HEADS = 12
_KEY_DIM = 384
_HEAD_DIM = _KEY_DIM // _NUM_HEADS
_BLOCK_SIZE = 32
_WINDOW_SIZE = 2
_NUM_RAND_BLOCKS = 2
_GLOBAL_TOKENS = 0
_D_MODEL = 768
_SEQ_LEN = 2048


def _bigbird_structure():
    seq_len, block_size = _SEQ_LEN, _BLOCK_SIZE
    num_blocks = (seq_len + block_size - 1) // block_size
    rows, cols = [], []
    for i in range(num_blocks):
        lo = max(0, i - _WINDOW_SIZE)
        hi = min(num_blocks, i + _WINDOW_SIZE + 1)
        for j in range(lo, hi):
            rows.append(i)
            cols.append(j)
    for i in range(num_blocks):
        for g in range(_GLOBAL_TOKENS):
            rows.append(i)
            cols.append(g)
    for g in range(_GLOBAL_TOKENS):
        for j in range(num_blocks):
            rows.append(g)
            cols.append(j)
    rng = np.random.default_rng(0)
    n_rand = _NUM_RAND_BLOCKS * num_blocks
    r1 = rng.integers(0, num_blocks, size=n_rand)
    r2 = rng.integers(0, num_blocks, size=n_rand)
    rows = np.concatenate([np.asarray(rows, dtype=np.int64), r1.astype(np.int64)])
    cols = np.concatenate([np.asarray(cols, dtype=np.int64), r2.astype(np.int64)])
    uniq = np.unique(rows * num_blocks + cols)
    ur = (uniq // num_blocks).astype(np.int64)
    uc = (uniq % num_blocks).astype(np.int64)
    nbr_lists = [uc[ur == b].astype(np.int32) for b in range(num_blocks)]
    max_nb = max(len(t) for t in nbr_lists)
    M = max_nb * block_size
    nbr_pad = np.zeros((num_blocks, max_nb), dtype=np.int32)
    bias = np.full((num_blocks, M), -1e9, dtype=np.float32)
    dense_idx = -np.ones((num_blocks, M), dtype=np.int32)
    for b, t in enumerate(nbr_lists):
        nbr_pad[b, : len(t)] = t
        bias[b, : len(t) * block_size] = 0.0
        if len(t):
            toks = np.concatenate(
                [np.arange(c * block_size, (c + 1) * block_size) for c in t])
            dense_idx[b, : len(toks)] = toks.astype(np.int32)
    token_block = (np.arange(seq_len) // block_size).astype(np.int64)
    attn_idx = dense_idx[token_block]
    return num_blocks, max_nb, M, nbr_pad, bias, attn_idx


_NUM_BLOCKS, _MAX_NB, _M, _NBR_PAD, _BIAS_NP, _ATTN_IDX_NP = _bigbird_structure()

_HEADS_PER_GROUP = 4
_NUM_GROUPS = _NUM_HEADS // _HEADS_PER_GROUP


_PROJ_STEPS = _SEQ_LEN // 256
_SLAB = 64
_HEADS_PER_SLAB = _SLAB // _HEAD_DIM
_NUM_SLABS = _KEY_DIM // _SLAB


def _mega_kernel(xq_ref, xk_ref, xv_ref, wq_ref, bq_ref, wk_ref, bk_ref,
                 wv_ref, bv_ref, bias_ref, wo_ref, bo_ref, scores_ref, out_ref,
                 q_sc, k_sc, v_sc, p_ref, ctx_ref):
    t = pl.program_id(0)

    @pl.when(t < _PROJ_STEPS)
    def _proj_phase():
        rows = pl.ds(t * 256, 256)
        qc = (jnp.dot(xq_ref[...], wq_ref[...], preferred_element_type=jnp.float32)
              + bq_ref[...]).astype(jnp.bfloat16)
        kc = (jnp.dot(xk_ref[...], wk_ref[...], preferred_element_type=jnp.float32)
              + bk_ref[...]).astype(jnp.bfloat16)
        vc = (jnp.dot(xv_ref[...], wv_ref[...], preferred_element_type=jnp.float32)
              + bv_ref[...]).astype(jnp.bfloat16)
        for gi in range(_NUM_SLABS):
            gl = slice(gi * _SLAB, (gi + 1) * _SLAB)
            q_sc[gi, rows, :] = qc[:, gl]
            k_sc[gi, rows, :] = kc[:, gl]
            v_sc[gi, rows, :] = vc[:, gl]

    @pl.when(t >= _PROJ_STEPS)
    def _attn_phase():
        g = t - _PROJ_STEPS
        _attn_body(g, q_sc[g], k_sc[g], v_sc[g], bias_ref, wo_ref, bo_ref,
                   scores_ref, out_ref, p_ref, ctx_ref)


def _attn_body(g, q128, k128, v128, bias_ref, wo_ref, bo_ref, scores_ref, out_ref,
               p_ref, ctx_ref):
    for hh in range(_HEADS_PER_SLAB):
        sl = slice(hh * _HEAD_DIM, (hh + 1) * _HEAD_DIM)
        q = q128[:, sl]
        k = k128[:, sl]
        v = v128[:, sl]
        for i in range(_NUM_BLOCKS):
            nbrs = _NBR_PAD[i]
            rows = slice(i * _BLOCK_SIZE, (i + 1) * _BLOCK_SIZE)
            q_i = q[rows, :]
            kn = jnp.concatenate(
                [k[int(c) * _BLOCK_SIZE:(int(c) + 1) * _BLOCK_SIZE, :] for c in nbrs],
                axis=0)
            s = jax.lax.dot_general(q_i, kn, (((1,), (1,)), ((), ())),
                                    preferred_element_type=jnp.float32)
            s = s + bias_ref[i:i + 1, :]
            scores_ref[0, hh, rows, :] = s
            m = jnp.max(s, axis=-1, keepdims=True)
            e = jnp.exp(s - m)
            p_ref[rows, :] = (e / jnp.sum(e, axis=-1, keepdims=True)).astype(jnp.bfloat16)
        ctx_parts = []
        for i in range(_NUM_BLOCKS):
            nbrs = _NBR_PAD[i]
            p_i = p_ref[i * _BLOCK_SIZE:(i + 1) * _BLOCK_SIZE, :]
            vn = jnp.concatenate(
                [v[int(c) * _BLOCK_SIZE:(int(c) + 1) * _BLOCK_SIZE, :] for c in nbrs],
                axis=0)
            ctx_parts.append(jax.lax.dot_general(p_i, vn, (((1,), (0,)), ((), ())),
                                                 preferred_element_type=jnp.float32))
        ctx_ref[g, :, sl] = jnp.concatenate(ctx_parts, axis=0).astype(jnp.bfloat16)

    @pl.when(g == _NUM_SLABS - 1)
    def _():
        acc = bo_ref[...].astype(jnp.float32)
        for gi in range(_NUM_SLABS):
            w_rows = slice(gi * _SLAB, (gi + 1) * _SLAB)
            acc = acc + jnp.dot(ctx_ref[gi], wo_ref[w_rows, :],
                                preferred_element_type=jnp.float32)
        out_ref[0] = acc


def kernel(query, value, key_in, Wq, bq, Wk, bk, Wv, bv, Wo, bo):
    B, S, D = query.shape
    H, hd, M, NB = _NUM_HEADS, _HEAD_DIM, _M, _NUM_BLOCKS
    scale = float(hd) ** -0.5

    ROWS = 256
    GL = _HEADS_PER_GROUP * hd
    P = _PROJ_STEPS
    row_spec = pl.BlockSpec((ROWS, D), lambda t: (jnp.minimum(t, P - 1), 0))
    w_spec = pl.BlockSpec((D, _KEY_DIM), lambda t: (0, 0))
    b_spec = pl.BlockSpec((1, _KEY_DIM), lambda t: (0, 0))
    scores, out = pl.pallas_call(
        _mega_kernel,
        grid=(P + _NUM_SLABS,),
        in_specs=[row_spec, row_spec, row_spec,
                  w_spec, b_spec, w_spec, b_spec, w_spec, b_spec,
                  pl.BlockSpec((NB, M), lambda t: (0, 0)),
                  pl.BlockSpec((_KEY_DIM, D), lambda t: (0, 0)),
                  pl.BlockSpec((1, D), lambda t: (0, 0))],
        out_specs=[pl.BlockSpec(
                       (1, _HEADS_PER_SLAB, S, M),
                       lambda t: (0, jnp.clip(t - P, 0, _NUM_SLABS - 1), 0, 0)),
                   pl.BlockSpec((1, S, D), lambda t: (0, 0, 0))],
        out_shape=[jax.ShapeDtypeStruct((1, H, S, M), jnp.float32),
                   jax.ShapeDtypeStruct((1, S, D), jnp.float32)],
        scratch_shapes=[pltpu.VMEM((_NUM_SLABS, S, _SLAB), jnp.bfloat16),
                        pltpu.VMEM((_NUM_SLABS, S, _SLAB), jnp.bfloat16),
                        pltpu.VMEM((_NUM_SLABS, S, _SLAB), jnp.bfloat16),
                        pltpu.VMEM((S, M), jnp.bfloat16),
                        pltpu.VMEM((_NUM_SLABS, S, _SLAB), jnp.bfloat16)],
    )(query[0], key_in[0], value[0],
      Wq * scale, (bq * scale)[None, :], Wk, bk[None, :], Wv, bv[None, :],
      jnp.asarray(_BIAS_NP), Wo.astype(jnp.bfloat16), bo[None, :])

    return (out, scores, jnp.asarray(_ATTN_IDX_NP))


# speedup vs baseline: 1.1066x; 1.0348x over previous
import numpy as np
import jax
import jax.numpy as jnp
from jax.experimental import pallas as pl
from jax.experimental.pallas import tpu as pltpu

_NUM_HEADS = 12
_KEY_DIM = 384
_HEAD_DIM = _KEY_DIM // _NUM_HEADS
_BLOCK_SIZE = 32
_WINDOW_SIZE = 2
_NUM_RAND_BLOCKS = 2
_GLOBAL_TOKENS = 0
_D_MODEL = 768
_SEQ_LEN = 2048


def _bigbird_structure():
    seq_len, block_size = _SEQ_LEN, _BLOCK_SIZE
    num_blocks = (seq_len + block_size - 1) // block_size
    rows, cols = [], []
    for i in range(num_blocks):
        lo = max(0, i - _WINDOW_SIZE)
        hi = min(num_blocks, i + _WINDOW_SIZE + 1)
        for j in range(lo, hi):
            rows.append(i)
            cols.append(j)
    for i in range(num_blocks):
        for g in range(_GLOBAL_TOKENS):
            rows.append(i)
            cols.append(g)
    for g in range(_GLOBAL_TOKENS):
        for j in range(num_blocks):
            rows.append(g)
            cols.append(j)
    rng = np.random.default_rng(0)
    n_rand = _NUM_RAND_BLOCKS * num_blocks
    r1 = rng.integers(0, num_blocks, size=n_rand)
    r2 = rng.integers(0, num_blocks, size=n_rand)
    rows = np.concatenate([np.asarray(rows, dtype=np.int64), r1.astype(np.int64)])
    cols = np.concatenate([np.asarray(cols, dtype=np.int64), r2.astype(np.int64)])
    uniq = np.unique(rows * num_blocks + cols)
    ur = (uniq // num_blocks).astype(np.int64)
    uc = (uniq % num_blocks).astype(np.int64)
    nbr_lists = [uc[ur == b].astype(np.int32) for b in range(num_blocks)]
    max_nb = max(len(t) for t in nbr_lists)
    M = max_nb * block_size
    nbr_pad = np.zeros((num_blocks, max_nb), dtype=np.int32)
    bias = np.full((num_blocks, M), -1e9, dtype=np.float32)
    dense_idx = -np.ones((num_blocks, M), dtype=np.int32)
    for b, t in enumerate(nbr_lists):
        nbr_pad[b, : len(t)] = t
        bias[b, : len(t) * block_size] = 0.0
        if len(t):
            toks = np.concatenate(
                [np.arange(c * block_size, (c + 1) * block_size) for c in t])
            dense_idx[b, : len(toks)] = toks.astype(np.int32)
    token_block = (np.arange(seq_len) // block_size).astype(np.int64)
    attn_idx = dense_idx[token_block]
    return num_blocks, max_nb, M, nbr_pad, bias, attn_idx


_NUM_BLOCKS, _MAX_NB, _M, _NBR_PAD, _BIAS_NP, _ATTN_IDX_NP = _bigbird_structure()

_HEADS_PER_GROUP = 4
_NUM_GROUPS = _NUM_HEADS // _HEADS_PER_GROUP


_PROJ_STEPS = _SEQ_LEN // 512


def _mega_kernel(xq_ref, xk_ref, xv_ref, wq_ref, bq_ref, wk_ref, bk_ref,
                 wv_ref, bv_ref, bias_ref, wo_ref, bo_ref, scores_ref, out_ref,
                 q_sc, k_sc, v_sc, p_ref, ctx_ref):
    t = pl.program_id(0)

    @pl.when(t < _PROJ_STEPS)
    def _proj_phase():
        rows = pl.ds(t * 512, 512)
        qc = (jnp.dot(xq_ref[...], wq_ref[...], preferred_element_type=jnp.float32)
              + bq_ref[...]).astype(jnp.bfloat16)
        kc = (jnp.dot(xk_ref[...], wk_ref[...], preferred_element_type=jnp.float32)
              + bk_ref[...]).astype(jnp.bfloat16)
        vc = (jnp.dot(xv_ref[...], wv_ref[...], preferred_element_type=jnp.float32)
              + bv_ref[...]).astype(jnp.bfloat16)
        for gi in range(_NUM_GROUPS):
            gl = slice(gi * 128, (gi + 1) * 128)
            q_sc[gi, rows, :] = qc[:, gl]
            k_sc[gi, rows, :] = kc[:, gl]
            v_sc[gi, rows, :] = vc[:, gl]

    @pl.when(t >= _PROJ_STEPS)
    def _attn_phase():
        g = t - _PROJ_STEPS
        _attn_body(g, q_sc[g], k_sc[g], v_sc[g], bias_ref, wo_ref, bo_ref,
                   scores_ref, out_ref, p_ref, ctx_ref)


def _attn_body(g, q128, k128, v128, bias_ref, wo_ref, bo_ref, scores_ref, out_ref,
               p_ref, ctx_ref):
    for hh in range(_HEADS_PER_GROUP):
        sl = slice(hh * _HEAD_DIM, (hh + 1) * _HEAD_DIM)
        q = q128[:, sl]
        k = k128[:, sl]
        v = v128[:, sl]
        for i in range(_NUM_BLOCKS):
            nbrs = _NBR_PAD[i]
            rows = slice(i * _BLOCK_SIZE, (i + 1) * _BLOCK_SIZE)
            q_i = q[rows, :]
            kn = jnp.concatenate(
                [k[int(c) * _BLOCK_SIZE:(int(c) + 1) * _BLOCK_SIZE, :] for c in nbrs],
                axis=0)
            s = jax.lax.dot_general(q_i, kn, (((1,), (1,)), ((), ())),
                                    preferred_element_type=jnp.float32)
            s = s + bias_ref[i:i + 1, :]
            scores_ref[0, hh, rows, :] = s
            m = jnp.max(s, axis=-1, keepdims=True)
            e = jnp.exp(s - m)
            p_ref[rows, :] = (e / jnp.sum(e, axis=-1, keepdims=True)).astype(jnp.bfloat16)
        ctx_parts = []
        for i in range(_NUM_BLOCKS):
            nbrs = _NBR_PAD[i]
            p_i = p_ref[i * _BLOCK_SIZE:(i + 1) * _BLOCK_SIZE, :]
            vn = jnp.concatenate(
                [v[int(c) * _BLOCK_SIZE:(int(c) + 1) * _BLOCK_SIZE, :] for c in nbrs],
                axis=0)
            ctx_parts.append(jax.lax.dot_general(p_i, vn, (((1,), (0,)), ((), ())),
                                                 preferred_element_type=jnp.float32))
        ctx_ref[g, :, sl] = jnp.concatenate(ctx_parts, axis=0).astype(jnp.bfloat16)

    @pl.when(g == _NUM_GROUPS - 1)
    def _():
        acc = bo_ref[...].astype(jnp.float32)
        for gi in range(_NUM_GROUPS):
            w_rows = slice(gi * _HEADS_PER_GROUP * _HEAD_DIM,
                           (gi + 1) * _HEADS_PER_GROUP * _HEAD_DIM)
            acc = acc + jnp.dot(ctx_ref[gi], wo_ref[w_rows, :],
                                preferred_element_type=jnp.float32)
        out_ref[0] = acc


def kernel(query, value, key_in, Wq, bq, Wk, bk, Wv, bv, Wo, bo):
    B, S, D = query.shape
    H, hd, M, NB = _NUM_HEADS, _HEAD_DIM, _M, _NUM_BLOCKS
    scale = float(hd) ** -0.5

    ROWS = 512
    GL = _HEADS_PER_GROUP * hd
    P = _PROJ_STEPS
    row_spec = pl.BlockSpec((ROWS, D), lambda t: (jnp.minimum(t, P - 1), 0))
    w_spec = pl.BlockSpec((D, _KEY_DIM), lambda t: (0, 0))
    b_spec = pl.BlockSpec((1, _KEY_DIM), lambda t: (0, 0))
    scores, out = pl.pallas_call(
        _mega_kernel,
        grid=(P + _NUM_GROUPS,),
        in_specs=[row_spec, row_spec, row_spec,
                  w_spec, b_spec, w_spec, b_spec, w_spec, b_spec,
                  pl.BlockSpec((NB, M), lambda t: (0, 0)),
                  pl.BlockSpec((_KEY_DIM, D), lambda t: (0, 0)),
                  pl.BlockSpec((1, D), lambda t: (0, 0))],
        out_specs=[pl.BlockSpec(
                       (1, _HEADS_PER_GROUP, S, M),
                       lambda t: (0, jnp.clip(t - P, 0, _NUM_GROUPS - 1), 0, 0)),
                   pl.BlockSpec((1, S, D), lambda t: (0, 0, 0))],
        out_shape=[jax.ShapeDtypeStruct((1, H, S, M), jnp.float32),
                   jax.ShapeDtypeStruct((1, S, D), jnp.float32)],
        scratch_shapes=[pltpu.VMEM((_NUM_GROUPS, S, GL), jnp.bfloat16),
                        pltpu.VMEM((_NUM_GROUPS, S, GL), jnp.bfloat16),
                        pltpu.VMEM((_NUM_GROUPS, S, GL), jnp.bfloat16),
                        pltpu.VMEM((S, M), jnp.bfloat16),
                        pltpu.VMEM((_NUM_GROUPS, S, GL), jnp.bfloat16)],
    )(query[0], key_in[0], value[0],
      Wq * scale, (bq * scale)[None, :], Wk, bk[None, :], Wv, bv[None, :],
      jnp.asarray(_BIAS_NP), Wo.astype(jnp.bfloat16), bo[None, :])

    return (out, scores, jnp.asarray(_ATTN_IDX_NP))
